# baseline (speedup 1.0000x reference)
import jax
import jax.numpy as jnp
from jax.experimental import pallas as pl
from jax.experimental.pallas import tpu as pltpu

_BN = 512
_KSPLIT = 4


def _matmul_kernel(x_ref, *refs):
    w_refs = refs[:_KSPLIT]
    b_ref = refs[_KSPLIT]
    o_ref = refs[_KSPLIT + 1]
    xb_ref = refs[_KSPLIT + 2]
    i = pl.program_id(0)

    @pl.when(i == 0)
    def _cast_x_once():
        xb_ref[...] = x_ref[...].astype(jnp.bfloat16)

    kp = x_ref.shape[1] // _KSPLIT
    acc = None
    for j, w_ref in enumerate(w_refs):
        wb = w_ref[...].astype(jnp.bfloat16)
        part = jax.lax.dot_general(
            xb_ref[:, j * kp:(j + 1) * kp], wb,
            dimension_numbers=(((1,), (1,)), ((), ())),
            preferred_element_type=jnp.float32,
        )
        acc = part if acc is None else acc + part
    o_ref[...] = acc + b_ref[...]


def kernel(x, W, b):
    M, K = x.shape
    N = W.shape[0]
    kp = K // _KSPLIT
    b2 = b.reshape(1, N)
    w_specs = [
        pl.BlockSpec((_BN, kp), lambda i, j=j: (i, j)) for j in range(_KSPLIT)
    ]
    out = pl.pallas_call(
        _matmul_kernel,
        grid=(N // _BN,),
        in_specs=[pl.BlockSpec((M, K), lambda i: (0, 0))]
        + w_specs
        + [pl.BlockSpec((1, _BN), lambda i: (0, i))],
        out_specs=pl.BlockSpec((M, _BN), lambda i: (0, i)),
        out_shape=jax.ShapeDtypeStruct((M, N), jnp.float32),
        scratch_shapes=[pltpu.VMEM((M, K), jnp.bfloat16)],
        compiler_params=pltpu.CompilerParams(
            dimension_semantics=("arbitrary",),
        ),
    )(x, *([W] * _KSPLIT), b2)
    return out


# speedup vs baseline: 1.0302x; 1.0302x over previous
import jax
import jax.numpy as jnp
from jax.experimental import pallas as pl
from jax.experimental.pallas import tpu as pltpu

_BN = 512
_KSPLIT = 4


def _start_tile(w_hbm, buf, sems, step):
    kp = w_hbm.shape[1] // _KSPLIT
    for j in range(_KSPLIT):
        pltpu.make_async_copy(
            w_hbm.at[pl.ds(step * _BN, _BN), pl.ds(j * kp, kp)],
            buf.at[:, pl.ds(j * kp, kp)],
            sems.at[j],
        ).start()


def _wait_tile(w_hbm, buf, sems, step):
    kp = w_hbm.shape[1] // _KSPLIT
    for j in range(_KSPLIT):
        pltpu.make_async_copy(
            w_hbm.at[pl.ds(step * _BN, _BN), pl.ds(j * kp, kp)],
            buf.at[:, pl.ds(j * kp, kp)],
            sems.at[j],
        ).wait()


def _matmul_kernel(x_ref, w_hbm, b_ref, o_ref, buf0, buf1, xb_ref,
                   sems0, sems1):
    i = pl.program_id(0)
    nsteps = pl.num_programs(0)

    @pl.when(i == 0)
    def _prologue():
        _start_tile(w_hbm, buf0, sems0, 0)
        xb_ref[...] = x_ref[...].astype(jnp.bfloat16)

    nxt = i + 1
    even_next = jax.lax.rem(nxt, 2) == 0

    @pl.when(jnp.logical_and(nxt < nsteps, even_next))
    def _issue_even():
        _start_tile(w_hbm, buf0, sems0, nxt)

    @pl.when(jnp.logical_and(nxt < nsteps, jnp.logical_not(even_next)))
    def _issue_odd():
        _start_tile(w_hbm, buf1, sems1, nxt)

    def _compute(buf, sems):
        _wait_tile(w_hbm, buf, sems, i)
        wb = buf[...].astype(jnp.bfloat16)
        acc = jax.lax.dot_general(
            xb_ref[...], wb,
            dimension_numbers=(((1,), (1,)), ((), ())),
            preferred_element_type=jnp.float32,
        )
        o_ref[...] = acc + b_ref[...]

    @pl.when(jax.lax.rem(i, 2) == 0)
    def _compute_even():
        _compute(buf0, sems0)

    @pl.when(jax.lax.rem(i, 2) == 1)
    def _compute_odd():
        _compute(buf1, sems1)


def kernel(x, W, b):
    M, K = x.shape
    N = W.shape[0]
    b2 = b.reshape(1, N)
    out = pl.pallas_call(
        _matmul_kernel,
        grid=(N // _BN,),
        in_specs=[
            pl.BlockSpec((M, K), lambda i: (0, 0)),
            pl.BlockSpec(memory_space=pl.ANY),
            pl.BlockSpec((1, _BN), lambda i: (0, i)),
        ],
        out_specs=pl.BlockSpec((M, _BN), lambda i: (0, i)),
        out_shape=jax.ShapeDtypeStruct((M, N), jnp.float32),
        scratch_shapes=[
            pltpu.VMEM((_BN, K), jnp.float32),
            pltpu.VMEM((_BN, K), jnp.float32),
            pltpu.VMEM((M, K), jnp.bfloat16),
            pltpu.SemaphoreType.DMA((_KSPLIT,)),
            pltpu.SemaphoreType.DMA((_KSPLIT,)),
        ],
        compiler_params=pltpu.CompilerParams(
            dimension_semantics=("arbitrary",),
        ),
    )(x, W, b2)
    return out
